# baseline (speedup 1.0000x reference)
import jax
import jax.numpy as jnp
from jax import lax
from jax.experimental import pallas as pl
from jax.experimental.pallas import tpu as pltpu
from jax.experimental.pallas import tpu_sc as plsc

S = 4
N = 100000
D = 128
G = 512

NC = 2
NS = 16
NW = NC * NS
GPW = G // NW
B = 128
NCH = D // 16
BI_SL = 6256
GR = G // 16


def _sc_body(x_hbm, bi_hbm, out_hbm,
             bi_buf, focc_v, big_v, buf0, buf1, acc_v, slots_sh,
             sem0, sem1):
    sid = lax.axis_index("s")
    cid = lax.axis_index("c")
    wid = sid * NC + cid
    base_g = wid * GPW

    iota = lax.iota(jnp.int32, 16)
    zero16 = jnp.zeros((16,), jnp.float32)
    NP1 = jnp.float32(N + 1)

    for k in range(GR):
        focc_v[k, pl.ds(0, 16)] = zero16

    d0 = jnp.minimum(sid * BI_SL, N - BI_SL)
    p0 = sid * BI_SL
    bi_buf[pl.ds(0, 16)] = jnp.full((16,), -1, jnp.int32)

    @pl.when(d0 > 0)
    def _():
        pltpu.sync_copy(bi_hbm.at[pl.ds(d0 - 16, 16)], bi_buf.at[pl.ds(0, 16)])

    pltpu.sync_copy(bi_hbm.at[pl.ds(d0, BI_SL)], bi_buf.at[pl.ds(16, BI_SL)])

    def scan_body(t, carry):
        lbase = 16 + t * 16
        cur = bi_buf[pl.ds(lbase, 16)]
        prev = plsc.load_gather(bi_buf, [lbase - 1 + iota])
        gpos = d0 + t * 16 + iota
        changed = (cur != prev) & (gpos >= p0)
        val = (gpos + 1).astype(jnp.float32)
        plsc.store_scatter(focc_v, [cur >> 4, cur & 15], val, mask=changed)
        return carry

    lax.fori_loop(0, BI_SL // 16, scan_body, 0, unroll=4)

    pltpu.sync_copy(focc_v, slots_sh.at[sid])
    plsc.subcore_barrier()
    pltpu.sync_copy(slots_sh, big_v)

    def slot_body(s2, carry):
        return tuple(
            carry[k] + big_v[s2, k, pl.ds(0, 16)] for k in range(GR)
        )

    rows = lax.fori_loop(0, NS, slot_body, (zero16,) * GR)
    for k in range(GR):
        focc_v[k, pl.ds(0, 16)] = rows[k]

    def sufmin_body(k, run):
        row = focc_v[k, pl.ds(0, 16)]
        a = jnp.where(row == 0.0, NP1, row)
        return jnp.minimum(run, jnp.min(a))

    run_after = lax.fori_loop(wid + 1, GR, sufmin_body, NP1)

    row_cb = focc_v[wid, pl.ds(0, 16)]
    a_cb = jnp.where(row_cb == 0.0, NP1, row_cb)
    offs_f = [None] * 17
    offs_f[16] = run_after
    cur_m = run_after
    for i in range(15, -1, -1):
        cur_m = jnp.minimum(a_cb[i], cur_m)
        offs_f[i] = cur_m
    r0s = [(offs_f[i] - 1.0).astype(jnp.int32) for i in range(17)]

    def z_body(i, carry):
        acc_v[pl.ds(i * 16, 16)] = zero16
        return carry

    lax.fori_loop(0, GPW * D * S // 16, z_body, 0)

    w0 = (r0s[0] // 8) * 8
    w1 = r0s[GPW]
    nblk = (w1 - w0 + (B - 1)) // B
    nhalf = (nblk + 1) // 2

    def dma_start(b):
        return jnp.minimum(w0 + b * B, N - B)

    def issue(b, buf, sem):
        return pltpu.async_copy(x_hbm.at[pl.ds(dma_start(b), B)], buf, sem)

    def wait(buf, sem):
        pltpu.make_async_copy(x_hbm.at[pl.ds(0, B)], buf, sem).wait()

    def compute_block(b, buf):
        d0b = dma_start(b)
        bstart = w0 + b * B
        for gl in range(GPW):
            lo = jnp.maximum(r0s[gl], bstart)
            hi = jnp.minimum(r0s[gl + 1], d0b + B)

            @pl.when(hi > lo)
            def _():
                def row_body(j, acc):
                    new = list(acc)
                    for c in range(NCH):
                        xv = buf[j, pl.ds(c * 16, 16)]
                        a1 = jnp.abs(xv)
                        a2 = xv * xv
                        a3 = a2 * a1
                        a4 = a2 * a2
                        new[c] = acc[c] + a1
                        new[NCH + c] = acc[NCH + c] + a2
                        new[2 * NCH + c] = acc[2 * NCH + c] + a3
                        new[3 * NCH + c] = acc[3 * NCH + c] + a4
                    return tuple(new)

                acc = plsc.parallel_loop(
                    lo - d0b, hi - d0b, unroll=2, carry=(zero16,) * (S * NCH)
                )(lambda j, a: row_body(j, a))
                iota4 = iota * S
                for c in range(NCH):
                    for s in range(S):
                        idx = iota4 + (gl * D * S + c * 16 * S + s)
                        plsc.addupdate_scatter(acc_v, [idx], acc[s * NCH + c])

    issue(0, buf0, sem0)

    def pair_body(q, carry):
        b0 = 2 * q
        issue(b0 + 1, buf1, sem1)
        wait(buf0, sem0)
        compute_block(b0, buf0)
        issue(b0 + 2, buf0, sem0)
        wait(buf1, sem1)
        compute_block(b0 + 1, buf1)
        return carry

    lax.fori_loop(0, nhalf, pair_body, 0)
    wait(buf0, sem0)

    for gl in range(GPW):
        pltpu.sync_copy(
            acc_v.at[pl.ds(gl * D * S, D * S)], out_hbm.at[base_g + gl]
        )


def kernel(x, batch_index):
    mesh = plsc.VectorSubcoreMesh(core_axis_name="c", subcore_axis_name="s")
    f = pl.kernel(
        _sc_body,
        mesh=mesh,
        out_type=jax.ShapeDtypeStruct((G, D * S), jnp.float32),
        scratch_types=[
            pltpu.VMEM((16 + BI_SL,), jnp.int32),
            pltpu.VMEM((GR, 16), jnp.float32),
            pltpu.VMEM((NS, GR, 16), jnp.float32),
            pltpu.VMEM((B, D), jnp.float32),
            pltpu.VMEM((B, D), jnp.float32),
            pltpu.VMEM((GPW * D * S,), jnp.float32),
            pltpu.VMEM_SHARED((NS, GR, 16), jnp.float32),
            pltpu.SemaphoreType.DMA,
            pltpu.SemaphoreType.DMA,
        ],
        compiler_params=pltpu.CompilerParams(needs_layout_passes=False),
    )
    return f(x, batch_index.astype(jnp.int32)).reshape(G, D, S)


# speedup vs baseline: 1.1667x; 1.1667x over previous
import jax
import jax.numpy as jnp
from jax import lax
from jax.experimental import pallas as pl
from jax.experimental.pallas import tpu as pltpu
from jax.experimental.pallas import tpu_sc as plsc

S = 4
N = 100000
D = 128
G = 512

NC = 2
NS = 16
NW = NC * NS
GPW = G // NW
B = 128
NCH = D // 16
BI_SL = 6256
GR = G // 16


def _sc_body(x_hbm, bi_hbm, out_hbm,
             bi_buf, focc_v, big_v, buf0, buf1, acc_v, slots_sh,
             sem0, sem1):
    sid = lax.axis_index("s")
    cid = lax.axis_index("c")
    wid = sid * NC + cid
    base_g = wid * GPW

    iota = lax.iota(jnp.int32, 16)
    zero16 = jnp.zeros((16,), jnp.float32)
    NP1 = jnp.float32(N + 1)

    for k in range(GR):
        focc_v[k, pl.ds(0, 16)] = zero16

    d0 = jnp.minimum(sid * BI_SL, N - BI_SL)
    p0 = sid * BI_SL
    bi_buf[pl.ds(0, 16)] = jnp.full((16,), -1, jnp.int32)

    @pl.when(d0 > 0)
    def _():
        pltpu.sync_copy(bi_hbm.at[pl.ds(d0 - 16, 16)], bi_buf.at[pl.ds(0, 16)])

    pltpu.sync_copy(bi_hbm.at[pl.ds(d0, BI_SL)], bi_buf.at[pl.ds(16, BI_SL)])

    def scan_body(t, carry):
        lbase = 16 + t * 16
        cur = bi_buf[pl.ds(lbase, 16)]
        prev = plsc.load_gather(bi_buf, [lbase - 1 + iota])
        gpos = d0 + t * 16 + iota
        changed = (cur != prev) & (gpos >= p0)
        val = (gpos + 1).astype(jnp.float32)
        plsc.store_scatter(focc_v, [cur >> 4, cur & 15], val, mask=changed)
        return carry

    lax.fori_loop(0, BI_SL // 16, scan_body, 0, unroll=4)

    pltpu.sync_copy(focc_v, slots_sh.at[sid])
    plsc.subcore_barrier()
    pltpu.sync_copy(slots_sh, big_v)

    def slot_body(s2, carry):
        return tuple(
            carry[k] + big_v[s2, k, pl.ds(0, 16)] for k in range(GR)
        )

    rows = lax.fori_loop(0, NS, slot_body, (zero16,) * GR)
    for k in range(GR):
        focc_v[k, pl.ds(0, 16)] = rows[k]

    def sufmin_body(k, run):
        row = focc_v[k, pl.ds(0, 16)]
        a = jnp.where(row == 0.0, NP1, row)
        return jnp.minimum(run, jnp.min(a))

    run_after = lax.fori_loop(wid + 1, GR, sufmin_body, NP1)

    row_cb = focc_v[wid, pl.ds(0, 16)]
    a_cb = jnp.where(row_cb == 0.0, NP1, row_cb)
    offs_f = [None] * 17
    offs_f[16] = run_after
    cur_m = run_after
    for i in range(15, -1, -1):
        cur_m = jnp.minimum(a_cb[i], cur_m)
        offs_f[i] = cur_m
    r0s = [(offs_f[i] - 1.0).astype(jnp.int32) for i in range(17)]

    def z_body(i, carry):
        acc_v[pl.ds(i * 16, 16)] = zero16
        return carry

    lax.fori_loop(0, GPW * D * S // 16, z_body, 0)

    w0 = (r0s[0] // 8) * 8
    w1 = r0s[GPW]
    nblk = (w1 - w0 + (B - 1)) // B
    nhalf = (nblk + 1) // 2

    def dma_start(b):
        return jnp.minimum(w0 + b * B, N - B)

    def issue(b, buf, sem):
        return pltpu.async_copy(x_hbm.at[pl.ds(dma_start(b), B)], buf, sem)

    def wait(buf, sem):
        pltpu.make_async_copy(x_hbm.at[pl.ds(0, B)], buf, sem).wait()

    def compute_block(b, buf):
        d0b = dma_start(b)
        bstart = w0 + b * B
        for gl in range(GPW):
            lo = jnp.maximum(r0s[gl], bstart)
            hi = jnp.minimum(r0s[gl + 1], d0b + B)

            @pl.when(hi > lo)
            def _():
                def row_body(j, acc):
                    new = list(acc)
                    for c in range(NCH):
                        xv = buf[j, pl.ds(c * 16, 16)]
                        a1 = jnp.abs(xv)
                        a2 = xv * xv
                        a3 = a2 * a1
                        a4 = a2 * a2
                        new[c] = acc[c] + a1
                        new[NCH + c] = acc[NCH + c] + a2
                        new[2 * NCH + c] = acc[2 * NCH + c] + a3
                        new[3 * NCH + c] = acc[3 * NCH + c] + a4
                    return tuple(new)

                acc = plsc.parallel_loop(
                    lo - d0b, hi - d0b, carry=(zero16,) * (S * NCH)
                )(lambda j, a: row_body(j, a))
                iota4 = iota * S
                for c in range(NCH):
                    for s in range(S):
                        idx = iota4 + (gl * D * S + c * 16 * S + s)
                        plsc.addupdate_scatter(acc_v, [idx], acc[s * NCH + c])

    issue(0, buf0, sem0)

    def pair_body(q, carry):
        b0 = 2 * q
        issue(b0 + 1, buf1, sem1)
        wait(buf0, sem0)
        compute_block(b0, buf0)
        issue(b0 + 2, buf0, sem0)
        wait(buf1, sem1)
        compute_block(b0 + 1, buf1)
        return carry

    lax.fori_loop(0, nhalf, pair_body, 0)
    wait(buf0, sem0)

    for gl in range(GPW):
        pltpu.sync_copy(
            acc_v.at[pl.ds(gl * D * S, D * S)], out_hbm.at[base_g + gl]
        )


def kernel(x, batch_index):
    mesh = plsc.VectorSubcoreMesh(core_axis_name="c", subcore_axis_name="s")
    f = pl.kernel(
        _sc_body,
        mesh=mesh,
        out_type=jax.ShapeDtypeStruct((G, D * S), jnp.float32),
        scratch_types=[
            pltpu.VMEM((16 + BI_SL,), jnp.int32),
            pltpu.VMEM((GR, 16), jnp.float32),
            pltpu.VMEM((NS, GR, 16), jnp.float32),
            pltpu.VMEM((B, D), jnp.float32),
            pltpu.VMEM((B, D), jnp.float32),
            pltpu.VMEM((GPW * D * S,), jnp.float32),
            pltpu.VMEM_SHARED((NS, GR, 16), jnp.float32),
            pltpu.SemaphoreType.DMA,
            pltpu.SemaphoreType.DMA,
        ],
        compiler_params=pltpu.CompilerParams(needs_layout_passes=False),
    )
    return f(x, batch_index.astype(jnp.int32)).reshape(G, D, S)
